# baseline (speedup 1.0000x reference)
import jax
import jax.numpy as jnp
from jax.experimental import pallas as pl
from jax.experimental.pallas import tpu as pltpu

_TOP_TEMP = 1.0


def _fused_body(xl_ref, xr_ref, p_ref, a_ref, b2_ref, scale_ref, out_ref,
                comb_ref):
    num_e = p_ref.shape[0]
    er = a_ref.shape[0]
    rank = er // num_e
    f32 = jnp.float32
    h = xl_ref.shape[1]
    nt = (((1,), (1,)), ((), ()))

    @pl.when(pl.program_id(0) == 0)
    def _():
        comb_ref[:er, :] = a_ref[...]
        comb_ref[er:, :] = p_ref[...]

    zz = (jax.lax.dot_general(xl_ref[...], comb_ref[:, :h], nt,
                              preferred_element_type=f32)
          + jax.lax.dot_general(xr_ref[...], comb_ref[:, h:], nt,
                                preferred_element_type=f32))
    z = zz[:, :er]
    sim = jnp.abs(zz[:, er:])

    row = jax.lax.broadcasted_iota(jnp.int32, (num_e, num_e), 0)
    col = jax.lax.broadcasted_iota(jnp.int32, (num_e, num_e), 1)
    lstrict = (row < col).astype(f32)

    m1 = jnp.max(sim, axis=1, keepdims=True)
    eq1 = (sim == m1).astype(f32)
    pre1 = jnp.dot(eq1, lstrict, preferred_element_type=f32)
    f1 = eq1 * (pre1 == 0.0)

    sim2 = jnp.where(f1 > 0.0, -jnp.inf, sim)
    m2 = jnp.max(sim2, axis=1, keepdims=True)
    eq2 = (sim2 == m2).astype(f32)
    pre2 = jnp.dot(eq2, lstrict, preferred_element_type=f32)
    f2 = eq2 * (pre2 == 0.0)

    e2 = jnp.exp((m2 - m1) / _TOP_TEMP)
    denom = 1.0 + e2
    scale = scale_ref[0, 0]
    coeff = (f1 + f2 * e2) * (scale / denom)

    erow = jax.lax.broadcasted_iota(jnp.int32, (num_e, er), 0)
    ecol = jax.lax.broadcasted_iota(jnp.int32, (num_e, er), 1)
    expand = (ecol // rank == erow).astype(f32)
    sel = jnp.dot(coeff, expand, preferred_element_type=f32)

    w = z * sel
    out_ref[...] = jnp.dot(w, b2_ref[...], preferred_element_type=f32)


def kernel(x, A_stack, B_stack, prototypes, scaling):
    batch = x.shape[0]
    rest = x.shape[1:-1]
    f_in = x.shape[-1]
    tok = x.reshape(-1, f_in)
    t = tok.shape[0]
    num_e, rank, _ = A_stack.shape
    f_out = B_stack.shape[1]
    er = num_e * rank

    a_flat = A_stack.reshape(er, f_in)
    b2 = B_stack.transpose(0, 2, 1).reshape(er, f_out)
    scale2 = scaling.reshape(1, 1)

    t_tile = 256
    grid = t // t_tile

    out = pl.pallas_call(
        _fused_body,
        grid=(grid,),
        in_specs=[
            pl.BlockSpec((t_tile, f_in // 2), lambda i: (i, 0)),
            pl.BlockSpec((t_tile, f_in // 2), lambda i: (i, 1)),
            pl.BlockSpec((num_e, f_in), lambda i: (0, 0)),
            pl.BlockSpec((er, f_in), lambda i: (0, 0)),
            pl.BlockSpec((er, f_out), lambda i: (0, 0)),
            pl.BlockSpec((1, 1), lambda i: (0, 0)),
        ],
        out_specs=pl.BlockSpec((t_tile, f_out), lambda i: (i, 0)),
        out_shape=jax.ShapeDtypeStruct((t, f_out), jnp.float32),
        scratch_shapes=[pltpu.VMEM((er + num_e, f_in), jnp.float32)],
    )(tok, tok, prototypes, a_flat, b2, scale2)

    return out.reshape((batch,) + rest + (f_out,))


# speedup vs baseline: 1.1658x; 1.1658x over previous
import jax
import jax.numpy as jnp
from jax.experimental import pallas as pl
from jax.experimental.pallas import tpu as pltpu

_TOP_TEMP = 1.0


def _fused_body(xl_ref, xr_ref, p_ref, a_ref, b2_ref, scale_ref, out_ref,
                comb_ref):
    num_e = p_ref.shape[0]
    er = a_ref.shape[0]
    rank = er // num_e
    f32 = jnp.float32
    h = xl_ref.shape[1]
    nt = (((1,), (1,)), ((), ()))

    @pl.when(pl.program_id(0) == 0)
    def _():
        comb_ref[:er, :] = a_ref[...]
        comb_ref[er:, :] = p_ref[...]

    zz = (jax.lax.dot_general(xl_ref[...], comb_ref[:, :h], nt,
                              preferred_element_type=f32)
          + jax.lax.dot_general(xr_ref[...], comb_ref[:, h:], nt,
                                preferred_element_type=f32))
    z = zz[:, :er]
    sim = jnp.abs(zz[:, er:])

    row = jax.lax.broadcasted_iota(jnp.int32, (num_e, num_e), 0)
    col = jax.lax.broadcasted_iota(jnp.int32, (num_e, num_e), 1)
    lstrict = (row < col).astype(f32)

    m1 = jnp.max(sim, axis=1, keepdims=True)
    eq1 = (sim == m1).astype(f32)
    pre1 = jnp.dot(eq1, lstrict, preferred_element_type=f32)
    f1 = eq1 * (pre1 == 0.0)

    sim2 = jnp.where(f1 > 0.0, -jnp.inf, sim)
    m2 = jnp.max(sim2, axis=1, keepdims=True)
    eq2 = (sim2 == m2).astype(f32)
    pre2 = jnp.dot(eq2, lstrict, preferred_element_type=f32)
    f2 = eq2 * (pre2 == 0.0)

    e2 = jnp.exp((m2 - m1) / _TOP_TEMP)
    denom = 1.0 + e2
    scale = scale_ref[0, 0]
    coeff = (f1 + f2 * e2) * (scale / denom)

    erow = jax.lax.broadcasted_iota(jnp.int32, (num_e, er), 0)
    ecol = jax.lax.broadcasted_iota(jnp.int32, (num_e, er), 1)
    expand = (ecol // rank == erow).astype(f32)
    sel = jnp.dot(coeff, expand, preferred_element_type=f32)

    w = z * sel
    out_ref[...] = jnp.dot(w, b2_ref[...], preferred_element_type=f32)


def kernel(x, A_stack, B_stack, prototypes, scaling):
    batch = x.shape[0]
    rest = x.shape[1:-1]
    f_in = x.shape[-1]
    tok = x.reshape(-1, f_in)
    t = tok.shape[0]
    num_e, rank, _ = A_stack.shape
    f_out = B_stack.shape[1]
    er = num_e * rank

    a_flat = A_stack.reshape(er, f_in)
    b2 = B_stack.transpose(0, 2, 1).reshape(er, f_out)
    scale2 = scaling.reshape(1, 1)

    t_tile = 512
    grid = t // t_tile

    out = pl.pallas_call(
        _fused_body,
        grid=(grid,),
        in_specs=[
            pl.BlockSpec((t_tile, f_in // 2), lambda i: (i, 0)),
            pl.BlockSpec((t_tile, f_in // 2), lambda i: (i, 1)),
            pl.BlockSpec((num_e, f_in), lambda i: (0, 0)),
            pl.BlockSpec((er, f_in), lambda i: (0, 0)),
            pl.BlockSpec((er, f_out), lambda i: (0, 0)),
            pl.BlockSpec((1, 1), lambda i: (0, 0)),
        ],
        out_specs=pl.BlockSpec((t_tile, f_out), lambda i: (i, 0)),
        out_shape=jax.ShapeDtypeStruct((t, f_out), jnp.float32),
        scratch_shapes=[pltpu.VMEM((er + num_e, f_in), jnp.float32)],
    )(tok, tok, prototypes, a_flat, b2, scale2)

    return out.reshape((batch,) + rest + (f_out,))


# speedup vs baseline: 1.1723x; 1.0056x over previous
import jax
import jax.numpy as jnp
from jax.experimental import pallas as pl
from jax.experimental.pallas import tpu as pltpu

_TOP_TEMP = 1.0


def _fused_body(xl_ref, xr_ref, p_ref, a_ref, b2_ref, scale_ref, out_ref,
                comb_ref):
    num_e = p_ref.shape[0]
    er = a_ref.shape[0]
    rank = er // num_e
    f32 = jnp.float32
    h = xl_ref.shape[1]
    nt = (((1,), (1,)), ((), ()))

    @pl.when(pl.program_id(0) == 0)
    def _():
        comb_ref[:er, :] = a_ref[...]
        comb_ref[er:, :] = p_ref[...]

    row = jax.lax.broadcasted_iota(jnp.int32, (num_e, num_e), 0)
    col = jax.lax.broadcasted_iota(jnp.int32, (num_e, num_e), 1)
    lstrict = (row < col).astype(f32)
    erow = jax.lax.broadcasted_iota(jnp.int32, (num_e, er), 0)
    ecol = jax.lax.broadcasted_iota(jnp.int32, (num_e, er), 1)
    expand = (ecol // rank == erow).astype(f32)
    scale = scale_ref[0, 0]

    t_rows = xl_ref.shape[0]
    half = t_rows // 2
    for hh in range(2):
        rows = pl.ds(hh * half, half)
        zz = (jax.lax.dot_general(xl_ref[rows, :], comb_ref[:, :h], nt,
                                  preferred_element_type=f32)
              + jax.lax.dot_general(xr_ref[rows, :], comb_ref[:, h:], nt,
                                    preferred_element_type=f32))
        z = zz[:, :er]
        sim = jnp.abs(zz[:, er:])

        m1 = jnp.max(sim, axis=1, keepdims=True)
        eq1 = (sim == m1).astype(f32)
        pre1 = jnp.dot(eq1, lstrict, preferred_element_type=f32)
        f1 = eq1 * (pre1 == 0.0)

        sim2 = jnp.where(f1 > 0.0, -jnp.inf, sim)
        m2 = jnp.max(sim2, axis=1, keepdims=True)
        eq2 = (sim2 == m2).astype(f32)
        pre2 = jnp.dot(eq2, lstrict, preferred_element_type=f32)
        f2 = eq2 * (pre2 == 0.0)

        e2 = jnp.exp((m2 - m1) / _TOP_TEMP)
        coeff = (f1 + f2 * e2) * (scale / (1.0 + e2))
        sel = jnp.dot(coeff, expand, preferred_element_type=f32)

        w = z * sel
        out_ref[rows, :] = jnp.dot(w, b2_ref[...],
                                   preferred_element_type=f32)


def kernel(x, A_stack, B_stack, prototypes, scaling):
    batch = x.shape[0]
    rest = x.shape[1:-1]
    f_in = x.shape[-1]
    tok = x.reshape(-1, f_in)
    t = tok.shape[0]
    num_e, rank, _ = A_stack.shape
    f_out = B_stack.shape[1]
    er = num_e * rank

    a_flat = A_stack.reshape(er, f_in)
    b2 = B_stack.transpose(0, 2, 1).reshape(er, f_out)
    scale2 = scaling.reshape(1, 1)

    t_tile = 512
    grid = t // t_tile

    out = pl.pallas_call(
        _fused_body,
        grid=(grid,),
        in_specs=[
            pl.BlockSpec((t_tile, f_in // 2), lambda i: (i, 0)),
            pl.BlockSpec((t_tile, f_in // 2), lambda i: (i, 1)),
            pl.BlockSpec((num_e, f_in), lambda i: (0, 0)),
            pl.BlockSpec((er, f_in), lambda i: (0, 0)),
            pl.BlockSpec((er, f_out), lambda i: (0, 0)),
            pl.BlockSpec((1, 1), lambda i: (0, 0)),
        ],
        out_specs=pl.BlockSpec((t_tile, f_out), lambda i: (i, 0)),
        out_shape=jax.ShapeDtypeStruct((t, f_out), jnp.float32),
        scratch_shapes=[pltpu.VMEM((er + num_e, f_in), jnp.float32)],
    )(tok, tok, prototypes, a_flat, b2, scale2)

    return out.reshape((batch,) + rest + (f_out,))
